# baseline (speedup 1.0000x reference)
import functools

import jax
import jax.numpy as jnp
import numpy as np
from jax import lax
from jax.experimental import pallas as pl
from jax.experimental.pallas import tpu as pltpu
from jax.experimental.pallas import tpu_sc as plsc

STEPS = 250
SPLIT = 66
STEPS_TC = STEPS - SPLIT
BETA = 0.02
NVAL = 10000
NITEMS = 4
SIZE = 3 * 128 * 128
NTILES = 32
BLOCKS_PER_ITEM = NTILES // NITEMS
BLK = SIZE // BLOCKS_PER_ITEM
NVEC = BLK // 16
ROWS, COLS = 384, 128
SUBROWS = ROWS // BLOCKS_PER_ITEM

_A = np.float32(np.sqrt(1.0 - BETA))
_B = np.float32(np.sqrt(BETA))
_MULT = np.uint32((1 << 32) % NVAL)


def _derive_keys():
    base = jax.random.key(42)
    item_keys = jax.vmap(lambda i: jax.random.fold_in(base, i))(jnp.arange(NITEMS))
    step_keys = jax.vmap(lambda k: jax.random.split(k, STEPS))(item_keys)
    sub = jax.vmap(jax.vmap(lambda k: jax.random.split(k)))(step_keys)
    kd = jax.random.key_data(sub)
    return kd.reshape(NITEMS, STEPS, 4)


def _tf_hash(k1, k2, ctr):
    kx = k1 ^ k2 ^ jnp.uint32(0x1BD11BDA)
    x0 = jnp.broadcast_to(k1, ctr.shape)
    x1 = ctr + k2

    def four(x0, x1, rots):
        for r in rots:
            x0 = x0 + x1
            x1 = (x1 << jnp.uint32(r)) | (x1 >> jnp.uint32(32 - r))
            x1 = x0 ^ x1
        return x0, x1

    ra = (13, 15, 26, 6)
    rb = (17, 29, 16, 24)
    x0, x1 = four(x0, x1, ra)
    x0 = x0 + k2
    x1 = x1 + kx + jnp.uint32(1)
    x0, x1 = four(x0, x1, rb)
    x0 = x0 + kx
    x1 = x1 + k1 + jnp.uint32(2)
    x0, x1 = four(x0, x1, ra)
    x0 = x0 + k1
    x1 = x1 + k2 + jnp.uint32(3)
    x0, x1 = four(x0, x1, rb)
    x0 = x0 + k2
    x1 = x1 + kx + jnp.uint32(4)
    x0, x1 = four(x0, x1, ra)
    x0 = x0 + kx
    x1 = x1 + k1 + jnp.uint32(5)
    return x0 ^ x1


def _rand_index_sc(hi_bits, lo_bits):
    s = jnp.uint32(NVAL)
    hi_m = hi_bits % s
    lo_m = lo_bits % s
    return plsc.bitcast((hi_m * _MULT + lo_m) % s, jnp.int32)


def _hash_body(x_hbm, dist_hbm, keys_hbm, out_hbm, table_v, keys_v, acc_v):
    wid = lax.axis_index("s") * 2 + lax.axis_index("c")
    item = wid // BLOCKS_PER_ITEM
    blk = wid % BLOCKS_PER_ITEM
    pltpu.sync_copy(dist_hbm.at[item], table_v)
    pltpu.sync_copy(keys_hbm.at[item], keys_v)
    pltpu.sync_copy(x_hbm.at[item, blk], acc_v)
    base_e = blk * BLK
    lane = lax.iota(jnp.int32, 16)

    def step_body(t, carry):
        kbase = t * 64
        k1a = keys_v[pl.ds(kbase, 16)]
        k1b = keys_v[pl.ds(kbase + 16, 16)]
        k2a = keys_v[pl.ds(kbase + 32, 16)]
        k2b = keys_v[pl.ds(kbase + 48, 16)]

        def inner(v, c):
            off = v * 16
            ctr = plsc.bitcast(base_e + off + lane, jnp.uint32)
            hi = _tf_hash(k1a, k1b, ctr)
            lo = _tf_hash(k2a, k2b, ctr)
            idx = _rand_index_sc(hi, lo)
            sample = plsc.load_gather(table_v, [idx])
            acc_v[pl.ds(off, 16)] = _A * acc_v[pl.ds(off, 16)] + _B * sample
            return c

        return lax.fori_loop(0, NVEC, inner, carry)

    lax.fori_loop(0, SPLIT, step_body, 0)
    pltpu.sync_copy(acc_v, out_hbm.at[item, blk])


def _mod_f32(x):
    r = (x >> jnp.uint32(16)) * jnp.uint32(5536) + (x & jnp.uint32(0xFFFF))
    ri = lax.bitcast_convert_type(r, jnp.int32)
    f = ri.astype(jnp.float32)
    q = (f * np.float32(1.0 / NVAL) + np.float32(0.5)).astype(jnp.int32)
    rem = ri - q * np.int32(NVAL)
    rem = jnp.where(rem < 0, rem + np.int32(NVAL), rem)
    return rem


def _idx_body(keys_ref, out_ref):
    e = (lax.broadcasted_iota(jnp.int32, (ROWS, COLS), 0) * COLS
         + lax.broadcasted_iota(jnp.int32, (ROWS, COLS), 1))
    ctr = lax.bitcast_convert_type(e, jnp.uint32)
    k1a, k1b, k2a, k2b = (jnp.uint32(keys_ref[0, 0, 0]),
                          jnp.uint32(keys_ref[0, 0, 1]),
                          jnp.uint32(keys_ref[0, 0, 2]),
                          jnp.uint32(keys_ref[0, 0, 3]))
    hi = _tf_hash(k1a, k1b, ctr)
    lo = _tf_hash(k2a, k2b, ctr)
    hi_m = _mod_f32(hi)
    lo_m = _mod_f32(lo)
    v = lax.bitcast_convert_type(hi_m * np.int32(_MULT) + lo_m, jnp.uint32)
    out_ref[0] = _mod_f32(v)


def _make_idx_kernel():
    return pl.pallas_call(
        _idx_body,
        grid=(NITEMS * STEPS_TC,),
        in_specs=[pl.BlockSpec((1, 1, 4), lambda g: (g, 0, 0),
                               memory_space=pltpu.SMEM)],
        out_specs=pl.BlockSpec((1, ROWS, COLS), lambda g: (g, 0, 0)),
        out_shape=jax.ShapeDtypeStruct((NITEMS * STEPS_TC, ROWS, COLS),
                                       jnp.int32),
    )


def _gather_body(xmid_hbm, dist_hbm, idx_hbm, out_hbm,
                 table_v, idx_v, acc_v, sem0, sem1, sem2, sem3):
    wid = lax.axis_index("s") * 2 + lax.axis_index("c")
    item = wid // BLOCKS_PER_ITEM
    blk = wid % BLOCKS_PER_ITEM
    row0 = item * STEPS_TC
    sems = (sem0, sem1, sem2, sem3)
    pltpu.sync_copy(dist_hbm.at[item], table_v)
    pltpu.sync_copy(xmid_hbm.at[item, blk], acc_v)

    def copy_in(t, buf):
        row = jnp.minimum(row0 + t, NITEMS * STEPS_TC - 1)
        return pltpu.make_async_copy(
            idx_hbm.at[row, pl.ds(blk * SUBROWS, SUBROWS)],
            idx_v.at[buf], sems[buf])

    for b in range(3):
        copy_in(b, b).start()

    def compute(buf):
        @plsc.parallel_loop(0, NVEC, 1, unroll=8)
        def _body(v):
            r = v >> 3
            cc = (v & 7) * 16
            iv = idx_v[buf, r, pl.ds(cc, 16)]
            sample = plsc.load_gather(table_v, [iv])
            off = v * 16
            acc_v[pl.ds(off, 16)] = _A * acc_v[pl.ds(off, 16)] + _B * sample

    def step4(tt, carry):
        t0 = tt * 4
        for b in range(4):
            copy_in(t0 + b, b).wait()
            copy_in(t0 + b + 3, (b + 3) % 4).start()
            compute(b)
        return carry

    lax.fori_loop(0, STEPS_TC // 4, step4, 0)
    for b in range(3):
        copy_in(0, b).wait()
    pltpu.sync_copy(acc_v, out_hbm.at[item, blk])


@jax.jit
def _run(x, dist, keys_sc, keys_tc):
    mesh = plsc.VectorSubcoreMesh(core_axis_name="c", subcore_axis_name="s")
    hash_k = pl.kernel(
        _hash_body,
        out_type=jax.ShapeDtypeStruct((NITEMS, BLOCKS_PER_ITEM, BLK),
                                      jnp.float32),
        mesh=mesh,
        scratch_types=[
            pltpu.VMEM((NVAL,), jnp.float32),
            pltpu.VMEM((SPLIT * 64,), jnp.uint32),
            pltpu.VMEM((BLK,), jnp.float32),
        ],
        compiler_params=pltpu.CompilerParams(needs_layout_passes=False),
    )
    xmid = hash_k(x, dist, keys_sc)
    idx = _make_idx_kernel()(keys_tc)
    gather_k = pl.kernel(
        _gather_body,
        out_type=jax.ShapeDtypeStruct((NITEMS, BLOCKS_PER_ITEM, BLK),
                                      jnp.float32),
        mesh=mesh,
        scratch_types=[
            pltpu.VMEM((NVAL,), jnp.float32),
            pltpu.VMEM((4, SUBROWS, COLS), jnp.int32),
            pltpu.VMEM((BLK,), jnp.float32),
            pltpu.SemaphoreType.DMA,
            pltpu.SemaphoreType.DMA,
            pltpu.SemaphoreType.DMA,
            pltpu.SemaphoreType.DMA,
        ],
        compiler_params=pltpu.CompilerParams(needs_layout_passes=False),
    )
    return gather_k(xmid, dist, idx)


def kernel(reflectance_normal, distribution):
    kd = _derive_keys()
    keys_sc = jnp.broadcast_to(
        kd[:, :SPLIT, :, None], (NITEMS, SPLIT, 4, 16)
    ).reshape(NITEMS, SPLIT * 64)
    keys_tc = lax.bitcast_convert_type(
        kd[:, SPLIT:].reshape(NITEMS * STEPS_TC, 1, 4), jnp.int32)
    x = reflectance_normal.reshape(NITEMS, BLOCKS_PER_ITEM, BLK)
    out = _run(x, distribution, keys_sc, keys_tc)
    return out.reshape(reflectance_normal.shape)


# speedup vs baseline: 1.0612x; 1.0612x over previous
import functools

import jax
import jax.numpy as jnp
import numpy as np
from jax import lax
from jax.experimental import pallas as pl
from jax.experimental.pallas import tpu as pltpu
from jax.experimental.pallas import tpu_sc as plsc

STEPS = 250
SPLIT = 70
STEPS_TC = STEPS - SPLIT
BETA = 0.02
NVAL = 10000
NITEMS = 4
SIZE = 3 * 128 * 128
NTILES = 32
BLOCKS_PER_ITEM = NTILES // NITEMS
BLK = SIZE // BLOCKS_PER_ITEM
NVEC = BLK // 16
ROWS, COLS = 384, 128
SUBROWS = ROWS // BLOCKS_PER_ITEM

_A = np.float32(np.sqrt(1.0 - BETA))
_B = np.float32(np.sqrt(BETA))
_MULT = np.uint32((1 << 32) % NVAL)


def _derive_keys():
    base = jax.random.key(42)
    item_keys = jax.vmap(lambda i: jax.random.fold_in(base, i))(jnp.arange(NITEMS))
    step_keys = jax.vmap(lambda k: jax.random.split(k, STEPS))(item_keys)
    sub = jax.vmap(jax.vmap(lambda k: jax.random.split(k)))(step_keys)
    kd = jax.random.key_data(sub)
    return kd.reshape(NITEMS, STEPS, 4)


def _tf_hash(k1, k2, ctr):
    kx = k1 ^ k2 ^ jnp.uint32(0x1BD11BDA)
    x0 = jnp.broadcast_to(k1, ctr.shape)
    x1 = ctr + k2

    def four(x0, x1, rots):
        for r in rots:
            x0 = x0 + x1
            x1 = (x1 << jnp.uint32(r)) | (x1 >> jnp.uint32(32 - r))
            x1 = x0 ^ x1
        return x0, x1

    ra = (13, 15, 26, 6)
    rb = (17, 29, 16, 24)
    x0, x1 = four(x0, x1, ra)
    x0 = x0 + k2
    x1 = x1 + kx + jnp.uint32(1)
    x0, x1 = four(x0, x1, rb)
    x0 = x0 + kx
    x1 = x1 + k1 + jnp.uint32(2)
    x0, x1 = four(x0, x1, ra)
    x0 = x0 + k1
    x1 = x1 + k2 + jnp.uint32(3)
    x0, x1 = four(x0, x1, rb)
    x0 = x0 + k2
    x1 = x1 + kx + jnp.uint32(4)
    x0, x1 = four(x0, x1, ra)
    x0 = x0 + kx
    x1 = x1 + k1 + jnp.uint32(5)
    return x0 ^ x1


def _rand_index_sc(hi_bits, lo_bits):
    s = jnp.uint32(NVAL)
    hi_m = hi_bits % s
    lo_m = lo_bits % s
    return plsc.bitcast((hi_m * _MULT + lo_m) % s, jnp.int32)


def _hash_body(x_hbm, dist_hbm, keys_hbm, out_hbm, table_v, keys_v, acc_v):
    wid = lax.axis_index("s") * 2 + lax.axis_index("c")
    item = wid // BLOCKS_PER_ITEM
    blk = wid % BLOCKS_PER_ITEM
    pltpu.sync_copy(dist_hbm.at[item], table_v)
    pltpu.sync_copy(keys_hbm.at[item], keys_v)
    pltpu.sync_copy(x_hbm.at[item, blk], acc_v)
    base_e = blk * BLK
    lane = lax.iota(jnp.int32, 16)

    def step_body(t, carry):
        kbase = t * 64
        k1a = keys_v[pl.ds(kbase, 16)]
        k1b = keys_v[pl.ds(kbase + 16, 16)]
        k2a = keys_v[pl.ds(kbase + 32, 16)]
        k2b = keys_v[pl.ds(kbase + 48, 16)]

        def inner(v, c):
            off = v * 16
            ctr = plsc.bitcast(base_e + off + lane, jnp.uint32)
            hi = _tf_hash(k1a, k1b, ctr)
            lo = _tf_hash(k2a, k2b, ctr)
            idx = _rand_index_sc(hi, lo)
            sample = plsc.load_gather(table_v, [idx])
            acc_v[pl.ds(off, 16)] = _A * acc_v[pl.ds(off, 16)] + _B * sample
            return c

        return lax.fori_loop(0, NVEC, inner, carry)

    lax.fori_loop(0, SPLIT, step_body, 0)
    pltpu.sync_copy(acc_v, out_hbm.at[item, blk])


def _approx_rem(x):
    r = (x >> jnp.uint32(16)) * jnp.uint32(5536) + (x & jnp.uint32(0xFFFF))
    ri = lax.bitcast_convert_type(r, jnp.int32)
    f = ri.astype(jnp.float32)
    q = (f * np.float32(1.0 / NVAL)).astype(jnp.int32)
    return ri - q * np.int32(NVAL)


def _final_mod(v):
    v2 = v + np.int32(330000)
    f = v2.astype(jnp.float32)
    q = (f * np.float32(1.0 / NVAL) + np.float32(0.5)).astype(jnp.int32)
    rem = v2 - q * np.int32(NVAL)
    return jnp.where(rem < 0, rem + np.int32(NVAL), rem)


TC_UNROLL = 2


def _idx_body(keys_ref, out_ref):
    e = (lax.broadcasted_iota(jnp.int32, (ROWS, COLS), 0) * COLS
         + lax.broadcasted_iota(jnp.int32, (ROWS, COLS), 1))
    ctr = lax.bitcast_convert_type(e, jnp.uint32)
    for u in range(TC_UNROLL):
        k1a, k1b, k2a, k2b = (jnp.uint32(keys_ref[u, 0, 0]),
                              jnp.uint32(keys_ref[u, 0, 1]),
                              jnp.uint32(keys_ref[u, 0, 2]),
                              jnp.uint32(keys_ref[u, 0, 3]))
        hi = _tf_hash(k1a, k1b, ctr)
        lo = _tf_hash(k2a, k2b, ctr)
        v = _approx_rem(hi) * np.int32(_MULT) + _approx_rem(lo)
        out_ref[u] = _final_mod(v)


def _make_idx_kernel():
    return pl.pallas_call(
        _idx_body,
        grid=(NITEMS * STEPS_TC // TC_UNROLL,),
        in_specs=[pl.BlockSpec((TC_UNROLL, 1, 4), lambda g: (g, 0, 0),
                               memory_space=pltpu.SMEM)],
        out_specs=pl.BlockSpec((TC_UNROLL, ROWS, COLS), lambda g: (g, 0, 0)),
        out_shape=jax.ShapeDtypeStruct((NITEMS * STEPS_TC, ROWS, COLS),
                                       jnp.int32),
    )


def _gather_body(xmid_hbm, dist_hbm, idx_hbm, out_hbm,
                 table_v, idx_v, acc_v, sem0, sem1, sem2, sem3):
    wid = lax.axis_index("s") * 2 + lax.axis_index("c")
    item = wid // BLOCKS_PER_ITEM
    blk = wid % BLOCKS_PER_ITEM
    row0 = item * STEPS_TC
    sems = (sem0, sem1, sem2, sem3)
    pltpu.sync_copy(dist_hbm.at[item], table_v)
    pltpu.sync_copy(xmid_hbm.at[item, blk], acc_v)

    def copy_in(t, buf):
        row = jnp.minimum(row0 + t, NITEMS * STEPS_TC - 1)
        return pltpu.make_async_copy(
            idx_hbm.at[row, pl.ds(blk * SUBROWS, SUBROWS)],
            idx_v.at[buf], sems[buf])

    for b in range(3):
        copy_in(b, b).start()

    def compute(buf):
        @plsc.parallel_loop(0, NVEC, 1, unroll=8)
        def _body(v):
            r = v >> 3
            cc = (v & 7) * 16
            iv = idx_v[buf, r, pl.ds(cc, 16)]
            sample = plsc.load_gather(table_v, [iv])
            off = v * 16
            acc_v[pl.ds(off, 16)] = _A * acc_v[pl.ds(off, 16)] + _B * sample

    def step4(tt, carry):
        t0 = tt * 4
        for b in range(4):
            copy_in(t0 + b, b).wait()
            copy_in(t0 + b + 3, (b + 3) % 4).start()
            compute(b)
        return carry

    lax.fori_loop(0, STEPS_TC // 4, step4, 0)
    for b in range(3):
        copy_in(0, b).wait()
    pltpu.sync_copy(acc_v, out_hbm.at[item, blk])


@jax.jit
def _run(x, dist, keys_sc, keys_tc):
    mesh = plsc.VectorSubcoreMesh(core_axis_name="c", subcore_axis_name="s")
    hash_k = pl.kernel(
        _hash_body,
        out_type=jax.ShapeDtypeStruct((NITEMS, BLOCKS_PER_ITEM, BLK),
                                      jnp.float32),
        mesh=mesh,
        scratch_types=[
            pltpu.VMEM((NVAL,), jnp.float32),
            pltpu.VMEM((SPLIT * 64,), jnp.uint32),
            pltpu.VMEM((BLK,), jnp.float32),
        ],
        compiler_params=pltpu.CompilerParams(needs_layout_passes=False),
    )
    xmid = hash_k(x, dist, keys_sc)
    idx = _make_idx_kernel()(keys_tc)
    gather_k = pl.kernel(
        _gather_body,
        out_type=jax.ShapeDtypeStruct((NITEMS, BLOCKS_PER_ITEM, BLK),
                                      jnp.float32),
        mesh=mesh,
        scratch_types=[
            pltpu.VMEM((NVAL,), jnp.float32),
            pltpu.VMEM((4, SUBROWS, COLS), jnp.int32),
            pltpu.VMEM((BLK,), jnp.float32),
            pltpu.SemaphoreType.DMA,
            pltpu.SemaphoreType.DMA,
            pltpu.SemaphoreType.DMA,
            pltpu.SemaphoreType.DMA,
        ],
        compiler_params=pltpu.CompilerParams(needs_layout_passes=False),
    )
    return gather_k(xmid, dist, idx)


def kernel(reflectance_normal, distribution):
    kd = _derive_keys()
    keys_sc = jnp.broadcast_to(
        kd[:, :SPLIT, :, None], (NITEMS, SPLIT, 4, 16)
    ).reshape(NITEMS, SPLIT * 64)
    keys_tc = lax.bitcast_convert_type(
        kd[:, SPLIT:].reshape(NITEMS * STEPS_TC, 1, 4), jnp.int32)
    x = reflectance_normal.reshape(NITEMS, BLOCKS_PER_ITEM, BLK)
    out = _run(x, distribution, keys_sc, keys_tc)
    return out.reshape(reflectance_normal.shape)
